# initial kernel scaffold (unmeasured)
import jax
import jax.numpy as jnp
from jax import lax
from jax.experimental import pallas as pl
from jax.experimental.pallas import tpu as pltpu

B, SQ, SKV, H, D = 4, 32, 4096, 8, 128
SCALE = D ** -0.5


def _flash_body(q_ref, k_ref, v_ref, o_ref, m_ref, l_ref):
    q = q_ref[0, :, 0, :]
    k = k_ref[0, :, 0, :]
    v = v_ref[0, :, 0, :]
    s = lax.dot_general(
        q, k, (((1,), (1,)), ((), ())), preferred_element_type=jnp.float32
    ) * SCALE
    m = jnp.max(s, axis=1, keepdims=True)
    p = jnp.exp(s - m)
    l = jnp.sum(p, axis=1, keepdims=True)
    o = lax.dot_general(
        p, v, (((1,), (0,)), ((), ())), preferred_element_type=jnp.float32
    )
    o_ref[0, :, 0, :] = o / l
    m_ref[0, 0, :, :] = m
    l_ref[0, 0, :, :] = l


def _local_flash(Q, K, V):
    return pl.pallas_call(
        _flash_body,
        grid=(B, H),
        in_specs=[
            pl.BlockSpec((1, SQ, 1, D), lambda b, h: (b, 0, h, 0)),
            pl.BlockSpec((1, SKV, 1, D), lambda b, h: (b, 0, h, 0)),
            pl.BlockSpec((1, SKV, 1, D), lambda b, h: (b, 0, h, 0)),
        ],
        out_shape=[
            jax.ShapeDtypeStruct((B, SQ, H, D), jnp.float32),
            jax.ShapeDtypeStruct((B, H, SQ, 1), jnp.float32),
            jax.ShapeDtypeStruct((B, H, SQ, 1), jnp.float32),
        ],
        out_specs=[
            pl.BlockSpec((1, SQ, 1, D), lambda b, h: (b, 0, h, 0)),
            pl.BlockSpec((1, 1, SQ, 1), lambda b, h: (b, h, 0, 0)),
            pl.BlockSpec((1, 1, SQ, 1), lambda b, h: (b, h, 0, 0)),
        ],
    )(Q, K, V)


def _combine_body(
    o_ref, m_ref, l_ref, out_ref, ro_ref, rm_ref, rl_ref, send_sems, recv_sems
):
    x = lax.axis_index("x")
    y = lax.axis_index("y")
    z = lax.axis_index("z")
    partner = (x, y, 1 - z)

    barrier = pltpu.get_barrier_semaphore()
    pl.semaphore_signal(
        barrier, inc=1, device_id=partner, device_id_type=pl.DeviceIdType.MESH
    )
    pl.semaphore_wait(barrier, 1)

    copies = []
    for i, (src, dst) in enumerate(
        ((o_ref, ro_ref), (m_ref, rm_ref), (l_ref, rl_ref))
    ):
        c = pltpu.make_async_remote_copy(
            src_ref=src,
            dst_ref=dst,
            send_sem=send_sems.at[i],
            recv_sem=recv_sems.at[i],
            device_id=partner,
            device_id_type=pl.DeviceIdType.MESH,
        )
        c.start()
        copies.append(c)
    for c in copies:
        c.wait()

    m_a = m_ref[...]
    m_b = rm_ref[...]
    m_n = jnp.maximum(m_a, m_b)
    a = jnp.exp(m_a - m_n) * l_ref[...]
    b = jnp.exp(m_b - m_n) * rl_ref[...]
    denom = a + b
    wa = a / denom
    wb = b / denom
    for bi in range(B):
        for h in range(H):
            out_ref[bi, :, h, :] = (
                o_ref[bi, :, h, :] * wa[bi, h] + ro_ref[bi, :, h, :] * wb[bi, h]
            )


def _combine(o_part, m, l):
    return pl.pallas_call(
        _combine_body,
        in_specs=[
            pl.BlockSpec(memory_space=pltpu.VMEM),
            pl.BlockSpec(memory_space=pltpu.VMEM),
            pl.BlockSpec(memory_space=pltpu.VMEM),
        ],
        out_shape=jax.ShapeDtypeStruct((B, SQ, H, D), jnp.float32),
        out_specs=pl.BlockSpec(memory_space=pltpu.VMEM),
        scratch_shapes=[
            pltpu.VMEM((B, SQ, H, D), jnp.float32),
            pltpu.VMEM((B, H, SQ, 1), jnp.float32),
            pltpu.VMEM((B, H, SQ, 1), jnp.float32),
            pltpu.SemaphoreType.DMA((3,)),
            pltpu.SemaphoreType.DMA((3,)),
        ],
        compiler_params=pltpu.CompilerParams(collective_id=0),
    )(o_part, m, l)


def kernel(Q, K, V):
    o_part, m, l = _local_flash(Q, K, V)
    return _combine(o_part, m, l)


# baseline (device time: 123151 ns/iter reference)
import jax
import jax.numpy as jnp
from jax import lax
from jax.experimental import pallas as pl
from jax.experimental.pallas import tpu as pltpu

B, SQ, SKV, H, D = 4, 32, 4096, 8, 128
SCALE = D ** -0.5


BKV = 1024
NKV = SKV // BKV


def _flash_body(q_ref, k_ref, v_ref, o_ref, m_ref, l_ref, acc_o, acc_m, acc_l):
    nk = pl.program_id(1)

    @pl.when(nk == 0)
    def _():
        acc_m[...] = jnp.full((H, SQ, 1), -jnp.inf, jnp.float32)
        acc_l[...] = jnp.zeros((H, SQ, 1), jnp.float32)
        acc_o[...] = jnp.zeros((SQ, H, D), jnp.float32)

    for h in range(H):
        q = q_ref[0, :, h, :]
        k = k_ref[0, :, h, :]
        v = v_ref[0, :, h, :]
        s = lax.dot_general(
            q, k, (((1,), (1,)), ((), ())), preferred_element_type=jnp.float32
        ) * SCALE
        m_prev = acc_m[h]
        m_blk = jnp.max(s, axis=1, keepdims=True)
        m_new = jnp.maximum(m_prev, m_blk)
        alpha = jnp.exp(m_prev - m_new)
        p = jnp.exp(s - m_new)
        l_new = acc_l[h] * alpha + jnp.sum(p, axis=1, keepdims=True)
        o_new = acc_o[:, h, :] * alpha + lax.dot_general(
            p, v, (((1,), (0,)), ((), ())), preferred_element_type=jnp.float32
        )
        acc_m[h] = m_new
        acc_l[h] = l_new
        acc_o[:, h, :] = o_new

    @pl.when(nk == NKV - 1)
    def _():
        for h in range(H):
            o_ref[0, :, h, :] = acc_o[:, h, :] / acc_l[h]
            m_ref[0, h] = acc_m[h]
            l_ref[0, h] = acc_l[h]


def _local_flash(Q, K, V):
    return pl.pallas_call(
        _flash_body,
        grid=(B, NKV),
        in_specs=[
            pl.BlockSpec((1, SQ, H, D), lambda b, nk: (b, 0, 0, 0)),
            pl.BlockSpec((1, BKV, H, D), lambda b, nk: (b, nk, 0, 0)),
            pl.BlockSpec((1, BKV, H, D), lambda b, nk: (b, nk, 0, 0)),
        ],
        out_shape=[
            jax.ShapeDtypeStruct((B, SQ, H, D), jnp.float32),
            jax.ShapeDtypeStruct((B, H, SQ, 1), jnp.float32),
            jax.ShapeDtypeStruct((B, H, SQ, 1), jnp.float32),
        ],
        out_specs=[
            pl.BlockSpec((1, SQ, H, D), lambda b, nk: (b, 0, 0, 0)),
            pl.BlockSpec((1, H, SQ, 1), lambda b, nk: (b, 0, 0, 0)),
            pl.BlockSpec((1, H, SQ, 1), lambda b, nk: (b, 0, 0, 0)),
        ],
        scratch_shapes=[
            pltpu.VMEM((SQ, H, D), jnp.float32),
            pltpu.VMEM((H, SQ, 1), jnp.float32),
            pltpu.VMEM((H, SQ, 1), jnp.float32),
        ],
    )(Q, K, V)


def _combine_body(
    o_ref, m_ref, l_ref, out_ref, ro_ref, rm_ref, rl_ref, send_sems, recv_sems
):
    x = lax.axis_index("x")
    y = lax.axis_index("y")
    z = lax.axis_index("z")
    partner = (x, y, 1 - z)

    barrier = pltpu.get_barrier_semaphore()
    pl.semaphore_signal(
        barrier, inc=1, device_id=partner, device_id_type=pl.DeviceIdType.MESH
    )
    pl.semaphore_wait(barrier, 1)

    copies = []
    for i, (src, dst) in enumerate(
        ((o_ref, ro_ref), (m_ref, rm_ref), (l_ref, rl_ref))
    ):
        c = pltpu.make_async_remote_copy(
            src_ref=src,
            dst_ref=dst,
            send_sem=send_sems.at[i],
            recv_sem=recv_sems.at[i],
            device_id=partner,
            device_id_type=pl.DeviceIdType.MESH,
        )
        c.start()
        copies.append(c)
    for c in copies:
        c.wait()

    m_a = m_ref[...]
    m_b = rm_ref[...]
    m_n = jnp.maximum(m_a, m_b)
    a = jnp.exp(m_a - m_n) * l_ref[...]
    b = jnp.exp(m_b - m_n) * rl_ref[...]
    denom = a + b
    wa = a / denom
    wb = b / denom
    for bi in range(B):
        for h in range(H):
            out_ref[bi, :, h, :] = (
                o_ref[bi, :, h, :] * wa[bi, h] + ro_ref[bi, :, h, :] * wb[bi, h]
            )


def _combine(o_part, m, l):
    return pl.pallas_call(
        _combine_body,
        in_specs=[
            pl.BlockSpec(memory_space=pltpu.VMEM),
            pl.BlockSpec(memory_space=pltpu.VMEM),
            pl.BlockSpec(memory_space=pltpu.VMEM),
        ],
        out_shape=jax.ShapeDtypeStruct((B, SQ, H, D), jnp.float32),
        out_specs=pl.BlockSpec(memory_space=pltpu.VMEM),
        scratch_shapes=[
            pltpu.VMEM((B, SQ, H, D), jnp.float32),
            pltpu.VMEM((B, H, SQ, 1), jnp.float32),
            pltpu.VMEM((B, H, SQ, 1), jnp.float32),
            pltpu.SemaphoreType.DMA((3,)),
            pltpu.SemaphoreType.DMA((3,)),
        ],
        compiler_params=pltpu.CompilerParams(collective_id=0),
    )(o_part, m, l)


def kernel(Q, K, V):
    o_part, m, l = _local_flash(Q, K, V)
    return _combine(o_part, m, l)
